# baseline (speedup 1.0000x reference)
import functools

import jax
import jax.numpy as jnp
from jax import lax
from jax.experimental import pallas as pl
from jax.experimental.pallas import tpu as pltpu
from jax.experimental.pallas import tpu_sc as plsc

N = 10000
E = 160000
D = 128
H = 128
C = 16

NC = 2
NS = 16
LANES = 128
ROWS_PER_TILE = 640
NPAD = NS * ROWS_PER_TILE
EPT = E // NS
KB = 16
NCHUNK = ((-(-EPT // LANES) + KB - 1) // KB) * KB
NBLK = NCHUNK // KB
EPT_PAD = NCHUNK * LANES
PADLEN = EPT_PAD * NS - E
NBUF = 2

_mesh = plsc.VectorSubcoreMesh(
    core_axis_name="c", subcore_axis_name="s", num_cores=NC, num_subcores=NS)


@functools.partial(
    pl.kernel,
    out_type=jax.ShapeDtypeStruct((NC, 2, NS, ROWS_PER_TILE, H), jnp.float32),
    mesh=_mesh,
    scratch_types=[
        pltpu.VMEM((NCHUNK, LANES), jnp.int32),
        pltpu.VMEM((LANES, H), jnp.float32),
        pltpu.VMEM_SHARED((NPAD, H), jnp.float32),
    ],
)
def _degree_kernel(idx_hbm, ones_hbm, zeros_hbm, out_hbm,
                   idx_v, ones_v, hist):
    c = lax.axis_index("c")
    s = lax.axis_index("s")
    base = s * ROWS_PER_TILE
    pltpu.sync_copy(ones_hbm, ones_v)
    for t in range(2):
        pltpu.sync_copy(zeros_hbm, hist.at[pl.ds(base, ROWS_PER_TILE)])
        pltpu.sync_copy(idx_hbm.at[c, t, s], idx_v)
        plsc.subcore_barrier()

        @pl.loop(0, NCHUNK)
        def _(j):
            pltpu.sync_copy(ones_v, hist.at[idx_v.at[j]], add=True)

        plsc.subcore_barrier()
        pltpu.sync_copy(hist.at[pl.ds(base, ROWS_PER_TILE)],
                        out_hbm.at[c, t, s])


@functools.partial(
    pl.kernel,
    out_type=jax.ShapeDtypeStruct((NC, NS, ROWS_PER_TILE, H), jnp.float32),
    mesh=_mesh,
    scratch_types=[
        pltpu.VMEM((NCHUNK, LANES), jnp.int32),
        pltpu.VMEM((NCHUNK, LANES), jnp.int32),
        pltpu.VMEM((LANES, H), jnp.float32),
        pltpu.VMEM_SHARED((NPAD, H), jnp.float32),
        pltpu.SemaphoreType.DMA,
    ],
)
def _propagate_kernel(y_hbm, src_hbm, dst_hbm, zeros_hbm, out_hbm,
                      src_v, dst_v, msg_v, agg, gsem):
    c = lax.axis_index("c")
    s = lax.axis_index("s")
    base = s * ROWS_PER_TILE
    pltpu.sync_copy(zeros_hbm, agg.at[pl.ds(base, ROWS_PER_TILE)])
    pltpu.sync_copy(src_hbm.at[c, s], src_v)
    pltpu.sync_copy(dst_hbm.at[c, s], dst_v)
    plsc.subcore_barrier()

    @pl.loop(0, NCHUNK)
    def _(j):
        pltpu.async_copy(y_hbm.at[src_v.at[j]], msg_v, gsem).wait()
        pltpu.sync_copy(msg_v, agg.at[dst_v.at[j]], add=True)

    plsc.subcore_barrier()
    pltpu.sync_copy(agg.at[pl.ds(base, ROWS_PER_TILE)], out_hbm.at[c, s])


R = 400
G = N // R


def _prep0_body(x_ref, do0_ref, do1_ref, w0_ref, w1_ref, y_ref):
    x = x_ref[...]
    r0 = lax.rsqrt(jnp.maximum(do0_ref[...], 1.0))
    r1 = lax.rsqrt(jnp.maximum(do1_ref[...], 1.0))
    y_ref[0] = jnp.dot(x * r0, w0_ref[...], preferred_element_type=jnp.float32)
    y_ref[1] = jnp.dot(x * r1, w1_ref[...], preferred_element_type=jnp.float32)


def _mid_body(a_ref, di0_ref, di1_ref, do0_ref, do1_ref,
              b0_ref, b1_ref, w0_ref, w1_ref, y_ref):
    ri0 = lax.rsqrt(jnp.maximum(di0_ref[...], 1.0))
    ri1 = lax.rsqrt(jnp.maximum(di1_ref[...], 1.0))
    h = jnp.maximum(
        a_ref[0] * ri0 + b0_ref[...] + a_ref[1] * ri1 + b1_ref[...], 0.0)
    ro0 = lax.rsqrt(jnp.maximum(do0_ref[...], 1.0))
    ro1 = lax.rsqrt(jnp.maximum(do1_ref[...], 1.0))
    y_ref[0] = jnp.dot(h * ro0, w0_ref[...], preferred_element_type=jnp.float32)
    y_ref[1] = jnp.dot(h * ro1, w1_ref[...], preferred_element_type=jnp.float32)


def _final_body(a_ref, di0_ref, di1_ref, b0_ref, b1_ref, wc_ref, bc_ref,
                out_ref, acc_ref):
    i = pl.program_id(0)
    ri0 = lax.rsqrt(jnp.maximum(di0_ref[...], 1.0))
    ri1 = lax.rsqrt(jnp.maximum(di1_ref[...], 1.0))
    h = jnp.maximum(
        a_ref[0] * ri0 + b0_ref[...] + a_ref[1] * ri1 + b1_ref[...], 0.0)

    @pl.when(i == 0)
    def _():
        acc_ref[...] = jnp.zeros_like(acc_ref)

    acc_ref[...] += jnp.sum(h, axis=0, keepdims=True)

    @pl.when(i == G - 1)
    def _():
        out_ref[...] = jnp.dot(acc_ref[...] * (1.0 / N), wc_ref[...],
                               preferred_element_type=jnp.float32) + bc_ref[...]


_row_spec = pl.BlockSpec((R, 1), lambda i: (i, 0))
_pair_in_spec = pl.BlockSpec((2, R, H), lambda i: (0, i, 0))
_pair_out_spec = pl.BlockSpec((2, R, H), lambda i: (0, i, 0))
_bias_spec = pl.BlockSpec((1, H), lambda i: (0, 0))
_w_spec = pl.BlockSpec((H, H), lambda i: (0, 0))


def _prep0(x, do0, do1, w0, w1):
    return pl.pallas_call(
        _prep0_body,
        grid=(G,),
        in_specs=[pl.BlockSpec((R, D), lambda i: (i, 0)),
                  _row_spec, _row_spec, _w_spec, _w_spec],
        out_specs=_pair_out_spec,
        out_shape=jax.ShapeDtypeStruct((2, NPAD, H), jnp.float32),
    )(x, do0, do1, w0, w1)


def _mid(a, di0, di1, do0, do1, b0, b1, w0, w1):
    return pl.pallas_call(
        _mid_body,
        grid=(G,),
        in_specs=[_pair_in_spec,
                  _row_spec, _row_spec, _row_spec, _row_spec,
                  _bias_spec, _bias_spec, _w_spec, _w_spec],
        out_specs=_pair_out_spec,
        out_shape=jax.ShapeDtypeStruct((2, NPAD, H), jnp.float32),
    )(a, di0, di1, do0, do1, b0, b1, w0, w1)


def _final(a, di0, di1, b0, b1, wc, bc):
    return pl.pallas_call(
        _final_body,
        grid=(G,),
        in_specs=[_pair_in_spec,
                  _row_spec, _row_spec,
                  _bias_spec, _bias_spec,
                  pl.BlockSpec((H, C), lambda i: (0, 0)),
                  pl.BlockSpec((1, C), lambda i: (0, 0))],
        out_specs=pl.BlockSpec((1, C), lambda i: (0, 0)),
        out_shape=jax.ShapeDtypeStruct((1, C), jnp.float32),
        scratch_shapes=[pltpu.VMEM((1, H), jnp.float32)],
    )(a, di0, di1, b0, b1, wc, bc)


def _pad_idx(v, val):
    return jnp.concatenate(
        [v, jnp.full((PADLEN,), val, jnp.int32)]).reshape(NS, NCHUNK, LANES)


def kernel(features, edge_index_r0, edge_index_r1,
           W0_r0, b0_r0, W0_r1, b0_r1,
           W1_r0, b1_r0, W1_r1, b1_r1,
           Wc, bc):
    s0, d0 = edge_index_r0[0], edge_index_r0[1]
    s1, d1 = edge_index_r1[0], edge_index_r1[1]

    deg_idx = jnp.stack([
        jnp.stack([_pad_idx(s0, N), _pad_idx(d0, N)]),
        jnp.stack([_pad_idx(s1, N), _pad_idx(d1, N)]),
    ])
    ones_in = jnp.ones((LANES, H), jnp.float32)
    zeros_h = jnp.zeros((ROWS_PER_TILE, H), jnp.float32)
    deg = _degree_kernel(deg_idx, ones_in, zeros_h)
    deg = deg.reshape(NC, 2, NPAD, H)
    do0 = deg[0, 0, :, 0:1]
    di0 = deg[0, 1, :, 0:1]
    do1 = deg[1, 0, :, 0:1]
    di1 = deg[1, 1, :, 0:1]

    src_prop = jnp.stack([_pad_idx(s0, N), _pad_idx(s1, N) + NPAD])
    dst_prop = jnp.stack([_pad_idx(d0, N), _pad_idx(d1, N)])

    b0_r0_ = b0_r0.reshape(1, H)
    b0_r1_ = b0_r1.reshape(1, H)
    b1_r0_ = b1_r0.reshape(1, H)
    b1_r1_ = b1_r1.reshape(1, H)
    bc_ = bc.reshape(1, C)

    y0 = _prep0(features, do0, do1, W0_r0, W0_r1)
    a0 = _propagate_kernel(y0.reshape(2 * NPAD, H), src_prop, dst_prop, zeros_h)
    a0 = a0.reshape(NC, NPAD, H)

    y1 = _mid(a0, di0, di1, do0, do1, b0_r0_, b0_r1_, W1_r0, W1_r1)
    a1 = _propagate_kernel(y1.reshape(2 * NPAD, H), src_prop, dst_prop, zeros_h)
    a1 = a1.reshape(NC, NPAD, H)

    return _final(a1, di0, di1, b1_r0_, b1_r1_, Wc, bc_)


# speedup vs baseline: 1.9244x; 1.9244x over previous
import functools

import jax
import jax.numpy as jnp
from jax import lax
from jax.experimental import pallas as pl
from jax.experimental.pallas import tpu as pltpu
from jax.experimental.pallas import tpu_sc as plsc

N = 10000
E = 160000
D = 128
H = 128
C = 16

NC = 2
NS = 16
LANES = 128
ROWS_PER_TILE = 640
NPAD = NS * ROWS_PER_TILE
EPT = E // NS
KB = 16
NCHUNK = ((-(-EPT // LANES) + KB - 1) // KB) * KB
NBLK = NCHUNK // KB
EPT_PAD = NCHUNK * LANES
PADLEN = EPT_PAD * NS - E
NBUF = 2

_mesh = plsc.VectorSubcoreMesh(
    core_axis_name="c", subcore_axis_name="s", num_cores=NC, num_subcores=NS)


@functools.partial(
    pl.kernel,
    out_type=jax.ShapeDtypeStruct((NC, 2, NS, ROWS_PER_TILE, H), jnp.float32),
    mesh=_mesh,
    scratch_types=[
        pltpu.VMEM((NCHUNK, LANES), jnp.int32),
        pltpu.VMEM((LANES, H), jnp.float32),
        pltpu.VMEM_SHARED((NPAD, H), jnp.float32),
    ],
)
def _degree_kernel(idx_hbm, ones_hbm, zeros_hbm, out_hbm,
                   idx_v, ones_v, hist):
    c = lax.axis_index("c")
    s = lax.axis_index("s")
    base = s * ROWS_PER_TILE
    pltpu.sync_copy(ones_hbm, ones_v)
    for t in range(2):
        pltpu.sync_copy(zeros_hbm, hist.at[pl.ds(base, ROWS_PER_TILE)])
        pltpu.sync_copy(idx_hbm.at[c, t, s], idx_v)
        plsc.subcore_barrier()

        @pl.loop(0, NCHUNK)
        def _(j):
            pltpu.sync_copy(ones_v, hist.at[idx_v.at[j]], add=True)

        plsc.subcore_barrier()
        pltpu.sync_copy(hist.at[pl.ds(base, ROWS_PER_TILE)],
                        out_hbm.at[c, t, s])


@functools.partial(
    pl.kernel,
    out_type=jax.ShapeDtypeStruct((NC, NS, ROWS_PER_TILE, H), jnp.float32),
    mesh=_mesh,
    scratch_types=[
        pltpu.VMEM((2, 2, KB, LANES), jnp.int32),
        pltpu.VMEM((NBUF, LANES, H), jnp.float32),
        pltpu.VMEM_SHARED((NPAD, H), jnp.float32),
        pltpu.SemaphoreType.DMA((2,)),
        pltpu.SemaphoreType.DMA,
        pltpu.SemaphoreType.DMA((NBUF,)),
    ],
)
def _propagate_kernel(y_hbm, idx_hbm, zeros_hbm, out_hbm,
                      idx_v, msg_v, agg, ibsem, gsem, ssem):
    c = lax.axis_index("c")
    s = lax.axis_index("s")
    base = s * ROWS_PER_TILE
    pltpu.sync_copy(zeros_hbm, agg.at[pl.ds(base, ROWS_PER_TILE)])
    pltpu.sync_copy(idx_hbm.at[c, s, 0], idx_v.at[0])
    plsc.subcore_barrier()
    pltpu.async_copy(idx_hbm.at[c, s, 1], idx_v.at[1], ibsem.at[1])

    @pl.loop(0, NCHUNK)
    def _(j):
        b = j & (NBUF - 1)
        m = j >> 4
        k = j & (KB - 1)
        ms = m & 1

        @pl.when((k == 0) & (j > 0))
        def _():
            pltpu.make_async_copy(
                idx_hbm.at[c, s, m], idx_v.at[ms], ibsem.at[ms]).wait()

        @pl.when(j >= NBUF)
        def _():
            pltpu.make_async_copy(
                msg_v.at[b], agg.at[idx_v.at[0, 1, 0]], ssem.at[b]).wait()

        @pl.when((k == 1) & (m + 1 < NBLK))
        def _():
            nxt = (m + 1) & 1
            pltpu.async_copy(idx_hbm.at[c, s, m + 1], idx_v.at[nxt],
                             ibsem.at[nxt])

        pltpu.async_copy(y_hbm.at[idx_v.at[ms, 0, k]], msg_v.at[b],
                         gsem).wait()
        pltpu.async_copy(msg_v.at[b], agg.at[idx_v.at[ms, 1, k]],
                         ssem.at[b], add=True)

    for b in range(NBUF):
        pltpu.make_async_copy(
            msg_v.at[b], agg.at[idx_v.at[0, 1, 0]], ssem.at[b]).wait()

    plsc.subcore_barrier()
    pltpu.sync_copy(agg.at[pl.ds(base, ROWS_PER_TILE)], out_hbm.at[c, s])


R = 400
G = N // R


def _prep0_body(x_ref, do0_ref, do1_ref, w0_ref, w1_ref, y_ref):
    x = x_ref[...]
    r0 = lax.rsqrt(jnp.maximum(do0_ref[...], 1.0))
    r1 = lax.rsqrt(jnp.maximum(do1_ref[...], 1.0))
    y_ref[0] = jnp.dot(x * r0, w0_ref[...], preferred_element_type=jnp.float32)
    y_ref[1] = jnp.dot(x * r1, w1_ref[...], preferred_element_type=jnp.float32)


def _mid_body(a_ref, di0_ref, di1_ref, do0_ref, do1_ref,
              b0_ref, b1_ref, w0_ref, w1_ref, y_ref):
    ri0 = lax.rsqrt(jnp.maximum(di0_ref[...], 1.0))
    ri1 = lax.rsqrt(jnp.maximum(di1_ref[...], 1.0))
    h = jnp.maximum(
        a_ref[0] * ri0 + b0_ref[...] + a_ref[1] * ri1 + b1_ref[...], 0.0)
    ro0 = lax.rsqrt(jnp.maximum(do0_ref[...], 1.0))
    ro1 = lax.rsqrt(jnp.maximum(do1_ref[...], 1.0))
    y_ref[0] = jnp.dot(h * ro0, w0_ref[...], preferred_element_type=jnp.float32)
    y_ref[1] = jnp.dot(h * ro1, w1_ref[...], preferred_element_type=jnp.float32)


def _final_body(a_ref, di0_ref, di1_ref, b0_ref, b1_ref, wc_ref, bc_ref,
                out_ref, acc_ref):
    i = pl.program_id(0)
    ri0 = lax.rsqrt(jnp.maximum(di0_ref[...], 1.0))
    ri1 = lax.rsqrt(jnp.maximum(di1_ref[...], 1.0))
    h = jnp.maximum(
        a_ref[0] * ri0 + b0_ref[...] + a_ref[1] * ri1 + b1_ref[...], 0.0)

    @pl.when(i == 0)
    def _():
        acc_ref[...] = jnp.zeros_like(acc_ref)

    acc_ref[...] += jnp.sum(h, axis=0, keepdims=True)

    @pl.when(i == G - 1)
    def _():
        out_ref[...] = jnp.dot(acc_ref[...] * (1.0 / N), wc_ref[...],
                               preferred_element_type=jnp.float32) + bc_ref[...]


_row_spec = pl.BlockSpec((R, 1), lambda i: (i, 0))
_pair_in_spec = pl.BlockSpec((2, R, H), lambda i: (0, i, 0))
_pair_out_spec = pl.BlockSpec((2, R, H), lambda i: (0, i, 0))
_bias_spec = pl.BlockSpec((1, H), lambda i: (0, 0))
_w_spec = pl.BlockSpec((H, H), lambda i: (0, 0))


def _prep0(x, do0, do1, w0, w1):
    return pl.pallas_call(
        _prep0_body,
        grid=(G,),
        in_specs=[pl.BlockSpec((R, D), lambda i: (i, 0)),
                  _row_spec, _row_spec, _w_spec, _w_spec],
        out_specs=_pair_out_spec,
        out_shape=jax.ShapeDtypeStruct((2, NPAD, H), jnp.float32),
    )(x, do0, do1, w0, w1)


def _mid(a, di0, di1, do0, do1, b0, b1, w0, w1):
    return pl.pallas_call(
        _mid_body,
        grid=(G,),
        in_specs=[_pair_in_spec,
                  _row_spec, _row_spec, _row_spec, _row_spec,
                  _bias_spec, _bias_spec, _w_spec, _w_spec],
        out_specs=_pair_out_spec,
        out_shape=jax.ShapeDtypeStruct((2, NPAD, H), jnp.float32),
    )(a, di0, di1, do0, do1, b0, b1, w0, w1)


def _final(a, di0, di1, b0, b1, wc, bc):
    return pl.pallas_call(
        _final_body,
        grid=(G,),
        in_specs=[_pair_in_spec,
                  _row_spec, _row_spec,
                  _bias_spec, _bias_spec,
                  pl.BlockSpec((H, C), lambda i: (0, 0)),
                  pl.BlockSpec((1, C), lambda i: (0, 0))],
        out_specs=pl.BlockSpec((1, C), lambda i: (0, 0)),
        out_shape=jax.ShapeDtypeStruct((1, C), jnp.float32),
        scratch_shapes=[pltpu.VMEM((1, H), jnp.float32)],
    )(a, di0, di1, b0, b1, wc, bc)


def _pad_idx(v):
    pad = N + jnp.arange(PADLEN, dtype=jnp.int32) % (NPAD - N)
    return jnp.concatenate([v, pad]).reshape(NS, NCHUNK, LANES)


def kernel(features, edge_index_r0, edge_index_r1,
           W0_r0, b0_r0, W0_r1, b0_r1,
           W1_r0, b1_r0, W1_r1, b1_r1,
           Wc, bc):
    s0, d0 = edge_index_r0[0], edge_index_r0[1]
    s1, d1 = edge_index_r1[0], edge_index_r1[1]

    p0, q0 = _pad_idx(s0), _pad_idx(d0)
    p1, q1 = _pad_idx(s1), _pad_idx(d1)
    deg_idx = jnp.stack([jnp.stack([p0, q0]), jnp.stack([p1, q1])])
    ones_in = jnp.ones((LANES, H), jnp.float32)
    zeros_h = jnp.zeros((ROWS_PER_TILE, H), jnp.float32)
    deg = _degree_kernel(deg_idx, ones_in, zeros_h)
    deg = deg.reshape(NC, 2, NPAD, H)
    do0 = deg[0, 0, :, 0:1]
    di0 = deg[0, 1, :, 0:1]
    do1 = deg[1, 0, :, 0:1]
    di1 = deg[1, 1, :, 0:1]

    src_prop = jnp.stack([p0, p1 + NPAD])
    dst_prop = jnp.stack([q0, q1])
    prop_idx = jnp.stack([
        src_prop.reshape(NC, NS, NBLK, KB, LANES),
        dst_prop.reshape(NC, NS, NBLK, KB, LANES),
    ], axis=3)

    b0_r0_ = b0_r0.reshape(1, H)
    b0_r1_ = b0_r1.reshape(1, H)
    b1_r0_ = b1_r0.reshape(1, H)
    b1_r1_ = b1_r1.reshape(1, H)
    bc_ = bc.reshape(1, C)

    y0 = _prep0(features, do0, do1, W0_r0, W0_r1)
    a0 = _propagate_kernel(y0.reshape(2 * NPAD, H), prop_idx, zeros_h)
    a0 = a0.reshape(NC, NPAD, H)

    y1 = _mid(a0, di0, di1, do0, do1, b0_r0_, b0_r1_, W1_r0, W1_r1)
    a1 = _propagate_kernel(y1.reshape(2 * NPAD, H), prop_idx, zeros_h)
    a1 = a1.reshape(NC, NPAD, H)

    return _final(a1, di0, di1, b1_r0_, b1_r1_, Wc, bc_)


# speedup vs baseline: 1.9294x; 1.0026x over previous
import functools

import jax
import jax.numpy as jnp
from jax import lax
from jax.experimental import pallas as pl
from jax.experimental.pallas import tpu as pltpu
from jax.experimental.pallas import tpu_sc as plsc

N = 10000
E = 160000
D = 128
H = 128
C = 16

NC = 2
NS = 16
LANES = 128
ROWS_PER_TILE = 640
NPAD = NS * ROWS_PER_TILE
EPT = E // NS
KB = 16
NCHUNK = ((-(-EPT // LANES) + KB - 1) // KB) * KB
NBLK = NCHUNK // KB
EPT_PAD = NCHUNK * LANES
PADLEN = EPT_PAD * NS - E
NBUF = 2
KD = 8

_mesh = plsc.VectorSubcoreMesh(
    core_axis_name="c", subcore_axis_name="s", num_cores=NC, num_subcores=NS)


@functools.partial(
    pl.kernel,
    out_type=jax.ShapeDtypeStruct((NC, 2, NS, ROWS_PER_TILE, H), jnp.float32),
    mesh=_mesh,
    scratch_types=[
        pltpu.VMEM((NCHUNK, LANES), jnp.int32),
        pltpu.VMEM((LANES, H), jnp.float32),
        pltpu.VMEM_SHARED((NPAD, H), jnp.float32),
        pltpu.SemaphoreType.DMA((KD,)),
    ],
)
def _degree_kernel(idx_hbm, ones_hbm, zeros_hbm, out_hbm,
                   idx_v, ones_v, hist, dsem):
    c = lax.axis_index("c")
    s = lax.axis_index("s")
    base = s * ROWS_PER_TILE
    pltpu.sync_copy(ones_hbm, ones_v)
    for t in range(2):
        pltpu.sync_copy(zeros_hbm, hist.at[pl.ds(base, ROWS_PER_TILE)])
        pltpu.sync_copy(idx_hbm.at[c, t, s], idx_v)
        plsc.subcore_barrier()

        @pl.loop(0, NCHUNK)
        def _(j):
            b = lax.rem(j, KD)

            @pl.when(j >= KD)
            def _():
                pltpu.make_async_copy(
                    ones_v, hist.at[idx_v.at[j]], dsem.at[b]).wait()

            pltpu.async_copy(ones_v, hist.at[idx_v.at[j]], dsem.at[b],
                             add=True)

        for b in range(min(KD, NCHUNK)):
            pltpu.make_async_copy(ones_v, hist.at[idx_v.at[0]],
                                  dsem.at[b]).wait()

        plsc.subcore_barrier()
        pltpu.sync_copy(hist.at[pl.ds(base, ROWS_PER_TILE)],
                        out_hbm.at[c, t, s])


@functools.partial(
    pl.kernel,
    out_type=jax.ShapeDtypeStruct((NC, NS, ROWS_PER_TILE, H), jnp.float32),
    mesh=_mesh,
    scratch_types=[
        pltpu.VMEM((2, 2, KB, LANES), jnp.int32),
        pltpu.VMEM((NBUF, LANES, H), jnp.float32),
        pltpu.VMEM_SHARED((NPAD, H), jnp.float32),
        pltpu.SemaphoreType.DMA((2,)),
        pltpu.SemaphoreType.DMA,
        pltpu.SemaphoreType.DMA((NBUF,)),
    ],
)
def _propagate_kernel(y_hbm, idx_hbm, zeros_hbm, out_hbm,
                      idx_v, msg_v, agg, ibsem, gsem, ssem):
    c = lax.axis_index("c")
    s = lax.axis_index("s")
    base = s * ROWS_PER_TILE
    pltpu.sync_copy(zeros_hbm, agg.at[pl.ds(base, ROWS_PER_TILE)])
    pltpu.sync_copy(idx_hbm.at[c, s, 0], idx_v.at[0])
    plsc.subcore_barrier()
    pltpu.async_copy(idx_hbm.at[c, s, 1], idx_v.at[1], ibsem.at[1])

    @pl.loop(0, NCHUNK)
    def _(j):
        b = j & (NBUF - 1)
        m = j >> 4
        k = j & (KB - 1)
        ms = m & 1

        @pl.when((k == 0) & (j > 0))
        def _():
            pltpu.make_async_copy(
                idx_hbm.at[c, s, m], idx_v.at[ms], ibsem.at[ms]).wait()

        @pl.when(j >= NBUF)
        def _():
            pltpu.make_async_copy(
                msg_v.at[b], agg.at[idx_v.at[0, 1, 0]], ssem.at[b]).wait()

        @pl.when((k == 1) & (m + 1 < NBLK))
        def _():
            nxt = (m + 1) & 1
            pltpu.async_copy(idx_hbm.at[c, s, m + 1], idx_v.at[nxt],
                             ibsem.at[nxt])

        pltpu.async_copy(y_hbm.at[idx_v.at[ms, 0, k]], msg_v.at[b],
                         gsem).wait()
        pltpu.async_copy(msg_v.at[b], agg.at[idx_v.at[ms, 1, k]],
                         ssem.at[b], add=True)

    for b in range(NBUF):
        pltpu.make_async_copy(
            msg_v.at[b], agg.at[idx_v.at[0, 1, 0]], ssem.at[b]).wait()

    plsc.subcore_barrier()
    pltpu.sync_copy(agg.at[pl.ds(base, ROWS_PER_TILE)], out_hbm.at[c, s])


R = 400
G = N // R


def _prep0_body(x_ref, do0_ref, do1_ref, w0_ref, w1_ref, y_ref):
    x = x_ref[...]
    r0 = lax.rsqrt(jnp.maximum(do0_ref[...], 1.0))
    r1 = lax.rsqrt(jnp.maximum(do1_ref[...], 1.0))
    y_ref[0] = jnp.dot(x * r0, w0_ref[...], preferred_element_type=jnp.float32)
    y_ref[1] = jnp.dot(x * r1, w1_ref[...], preferred_element_type=jnp.float32)


def _mid_body(a_ref, di0_ref, di1_ref, do0_ref, do1_ref,
              b0_ref, b1_ref, w0_ref, w1_ref, y_ref):
    ri0 = lax.rsqrt(jnp.maximum(di0_ref[...], 1.0))
    ri1 = lax.rsqrt(jnp.maximum(di1_ref[...], 1.0))
    h = jnp.maximum(
        a_ref[0] * ri0 + b0_ref[...] + a_ref[1] * ri1 + b1_ref[...], 0.0)
    ro0 = lax.rsqrt(jnp.maximum(do0_ref[...], 1.0))
    ro1 = lax.rsqrt(jnp.maximum(do1_ref[...], 1.0))
    y_ref[0] = jnp.dot(h * ro0, w0_ref[...], preferred_element_type=jnp.float32)
    y_ref[1] = jnp.dot(h * ro1, w1_ref[...], preferred_element_type=jnp.float32)


def _final_body(a_ref, di0_ref, di1_ref, b0_ref, b1_ref, wc_ref, bc_ref,
                out_ref, acc_ref):
    i = pl.program_id(0)
    ri0 = lax.rsqrt(jnp.maximum(di0_ref[...], 1.0))
    ri1 = lax.rsqrt(jnp.maximum(di1_ref[...], 1.0))
    h = jnp.maximum(
        a_ref[0] * ri0 + b0_ref[...] + a_ref[1] * ri1 + b1_ref[...], 0.0)

    @pl.when(i == 0)
    def _():
        acc_ref[...] = jnp.zeros_like(acc_ref)

    acc_ref[...] += jnp.sum(h, axis=0, keepdims=True)

    @pl.when(i == G - 1)
    def _():
        out_ref[...] = jnp.dot(acc_ref[...] * (1.0 / N), wc_ref[...],
                               preferred_element_type=jnp.float32) + bc_ref[...]


_row_spec = pl.BlockSpec((R, 1), lambda i: (i, 0))
_pair_in_spec = pl.BlockSpec((2, R, H), lambda i: (0, i, 0))
_pair_out_spec = pl.BlockSpec((2, R, H), lambda i: (0, i, 0))
_bias_spec = pl.BlockSpec((1, H), lambda i: (0, 0))
_w_spec = pl.BlockSpec((H, H), lambda i: (0, 0))


def _prep0(x, do0, do1, w0, w1):
    return pl.pallas_call(
        _prep0_body,
        grid=(G,),
        in_specs=[pl.BlockSpec((R, D), lambda i: (i, 0)),
                  _row_spec, _row_spec, _w_spec, _w_spec],
        out_specs=_pair_out_spec,
        out_shape=jax.ShapeDtypeStruct((2, NPAD, H), jnp.float32),
    )(x, do0, do1, w0, w1)


def _mid(a, di0, di1, do0, do1, b0, b1, w0, w1):
    return pl.pallas_call(
        _mid_body,
        grid=(G,),
        in_specs=[_pair_in_spec,
                  _row_spec, _row_spec, _row_spec, _row_spec,
                  _bias_spec, _bias_spec, _w_spec, _w_spec],
        out_specs=_pair_out_spec,
        out_shape=jax.ShapeDtypeStruct((2, NPAD, H), jnp.float32),
    )(a, di0, di1, do0, do1, b0, b1, w0, w1)


def _final(a, di0, di1, b0, b1, wc, bc):
    return pl.pallas_call(
        _final_body,
        grid=(G,),
        in_specs=[_pair_in_spec,
                  _row_spec, _row_spec,
                  _bias_spec, _bias_spec,
                  pl.BlockSpec((H, C), lambda i: (0, 0)),
                  pl.BlockSpec((1, C), lambda i: (0, 0))],
        out_specs=pl.BlockSpec((1, C), lambda i: (0, 0)),
        out_shape=jax.ShapeDtypeStruct((1, C), jnp.float32),
        scratch_shapes=[pltpu.VMEM((1, H), jnp.float32)],
    )(a, di0, di1, b0, b1, wc, bc)


def _pad_idx(v):
    pad = N + jnp.arange(PADLEN, dtype=jnp.int32) % (NPAD - N)
    return jnp.concatenate([v, pad]).reshape(NS, NCHUNK, LANES)


def kernel(features, edge_index_r0, edge_index_r1,
           W0_r0, b0_r0, W0_r1, b0_r1,
           W1_r0, b1_r0, W1_r1, b1_r1,
           Wc, bc):
    s0, d0 = edge_index_r0[0], edge_index_r0[1]
    s1, d1 = edge_index_r1[0], edge_index_r1[1]

    p0, q0 = _pad_idx(s0), _pad_idx(d0)
    p1, q1 = _pad_idx(s1), _pad_idx(d1)
    deg_idx = jnp.stack([jnp.stack([p0, q0]), jnp.stack([p1, q1])])
    ones_in = jnp.ones((LANES, H), jnp.float32)
    zeros_h = jnp.zeros((ROWS_PER_TILE, H), jnp.float32)
    deg = _degree_kernel(deg_idx, ones_in, zeros_h)
    deg = deg.reshape(NC, 2, NPAD, H)
    do0 = deg[0, 0, :, 0:1]
    di0 = deg[0, 1, :, 0:1]
    do1 = deg[1, 0, :, 0:1]
    di1 = deg[1, 1, :, 0:1]

    src_prop = jnp.stack([p0, p1 + NPAD])
    dst_prop = jnp.stack([q0, q1])
    prop_idx = jnp.stack([
        src_prop.reshape(NC, NS, NBLK, KB, LANES),
        dst_prop.reshape(NC, NS, NBLK, KB, LANES),
    ], axis=3)

    b0_r0_ = b0_r0.reshape(1, H)
    b0_r1_ = b0_r1.reshape(1, H)
    b1_r0_ = b1_r0.reshape(1, H)
    b1_r1_ = b1_r1.reshape(1, H)
    bc_ = bc.reshape(1, C)

    y0 = _prep0(features, do0, do1, W0_r0, W0_r1)
    a0 = _propagate_kernel(y0.reshape(2 * NPAD, H), prop_idx, zeros_h)
    a0 = a0.reshape(NC, NPAD, H)

    y1 = _mid(a0, di0, di1, do0, do1, b0_r0_, b0_r1_, W1_r0, W1_r1)
    a1 = _propagate_kernel(y1.reshape(2 * NPAD, H), prop_idx, zeros_h)
    a1 = a1.reshape(NC, NPAD, H)

    return _final(a1, di0, di1, b1_r0_, b1_r1_, Wc, bc_)
